# Initial kernel scaffold; baseline (speedup 1.0000x reference)
import functools

import jax
import jax.numpy as jnp
from jax import lax
from jax.experimental import pallas as pl
from jax.experimental.pallas import tpu as pltpu
from jax.experimental.pallas import tpu_sc as plsc

N = 10000
D = 128
E = 320000
G = 16
HID = 128
OUT = 128

NC = 2
NS = 16
NW = NC * NS
EPT = E // NW
EPC = E // NC
K = 80
NCH = EPT // K
RPT = N // NS

_MESH = plsc.VectorSubcoreMesh(core_axis_name="c", subcore_axis_name="s")


@functools.partial(
    pl.kernel,
    out_type=jax.ShapeDtypeStruct((NW, N), jnp.float32),
    mesh=_MESH,
    scratch_types=[
        pltpu.VMEM((EPT,), jnp.int32),
        pltpu.VMEM((N,), jnp.float32),
    ],
)
def _deg_kernel(dst_hbm, zeros_hbm, out_hbm, idx_v, hist_v):
    c = lax.axis_index("c")
    s = lax.axis_index("s")
    wid = s * NC + c
    pltpu.sync_copy(zeros_hbm, hist_v)
    pltpu.sync_copy(dst_hbm.at[pl.ds(wid * EPT, EPT)], idx_v)
    ones = jnp.full((16,), 1.0, dtype=jnp.float32)

    @pl.loop(0, EPT // 16)
    def _(j):
        idx16 = idx_v[pl.ds(j * 16, 16)]
        plsc.addupdate_scatter(hist_v, [idx16], ones)

    pltpu.sync_copy(hist_v, out_hbm.at[wid])


@functools.partial(
    pl.kernel,
    out_type=jax.ShapeDtypeStruct((NC, N, HID), jnp.float32),
    mesh=_MESH,
    scratch_types=[
        pltpu.VMEM((K,), jnp.int32),
        pltpu.VMEM((1, K), jnp.int32),
        pltpu.VMEM((K, HID), jnp.float32),
        pltpu.VMEM_SHARED((N, HID), jnp.float32),
        pltpu.SemaphoreType.DMA,
    ],
)
def _agg_kernel(ht_hbm, src_hbm, dst_hbm, zrows_hbm, out_hbm,
                sidx, didx, rows, acc, sem):
    c = lax.axis_index("c")
    s = lax.axis_index("s")
    base = c * EPC + s * EPT
    pltpu.sync_copy(zrows_hbm, acc.at[pl.ds(s * RPT, RPT)])
    plsc.subcore_barrier()

    @pl.loop(0, NCH)
    def _(i):
        eb = base + i * K
        pltpu.sync_copy(src_hbm.at[pl.ds(eb, K)], sidx)
        pltpu.sync_copy(dst_hbm.at[pl.ds(eb, K)], didx.at[0])
        pltpu.async_copy(ht_hbm.at[sidx], rows, sem).wait()
        pltpu.sync_copy(rows, acc.at[didx.at[0]], add=True)

    plsc.subcore_barrier()
    pltpu.sync_copy(acc.at[pl.ds(s * RPT, RPT)],
                    out_hbm.at[c, pl.ds(s * RPT, RPT)])


BR = 400
GRID = N // BR


def _mm1_body(hists_ref, x_ref, w1_ref, ht_ref, dinv_ref):
    deg = jnp.sum(hists_ref[...], axis=0) + 1.0
    dinv = lax.rsqrt(deg)
    h = jnp.dot(x_ref[...], w1_ref[...], preferred_element_type=jnp.float32)
    ht_ref[...] = h * dinv[:, None]
    dinv_ref[...] = dinv[:, None]


def _mid_body(aggp_ref, ht1_ref, dinv_ref, b1_ref, w2_ref, ht2_ref):
    dinv = dinv_ref[...]
    a = aggp_ref[...]
    m = dinv * (a[0] + a[1] + ht1_ref[...]) + b1_ref[...]
    h1r = jnp.maximum(m, 0.0)
    ht2_ref[...] = jnp.dot(
        h1r, w2_ref[...], preferred_element_type=jnp.float32) * dinv


def _pool_body(aggp_ref, ht2_ref, dinv_ref, b2_ref, batch_ref, wh_ref,
               bh_ref, out_ref, psum, pcnt):
    i = pl.program_id(0)

    @pl.when(i == 0)
    def _():
        psum[...] = jnp.zeros_like(psum)
        pcnt[...] = jnp.zeros_like(pcnt)

    a = aggp_ref[...]
    m = dinv_ref[...] * (a[0] + a[1] + ht2_ref[...]) + b2_ref[...]
    gids = lax.broadcasted_iota(jnp.int32, (G, BR), 0)
    oh = (gids == batch_ref[...]).astype(jnp.float32)
    psum[...] += jnp.dot(oh, m, preferred_element_type=jnp.float32)
    pcnt[...] += jnp.sum(oh, axis=1, keepdims=True)

    @pl.when(i == GRID - 1)
    def _():
        pooled = psum[...] / jnp.maximum(pcnt[...], 1.0)
        out_ref[...] = jnp.dot(
            pooled, wh_ref[...], preferred_element_type=jnp.float32) + bh_ref[...]


def _tc_mm1(hists, x, W1):
    return pl.pallas_call(
        _mm1_body,
        grid=(GRID,),
        in_specs=[
            pl.BlockSpec((NW, BR), lambda i: (0, i)),
            pl.BlockSpec((BR, D), lambda i: (i, 0)),
            pl.BlockSpec((D, HID), lambda i: (0, 0)),
        ],
        out_specs=[
            pl.BlockSpec((BR, HID), lambda i: (i, 0)),
            pl.BlockSpec((BR, 1), lambda i: (i, 0)),
        ],
        out_shape=[
            jax.ShapeDtypeStruct((N, HID), jnp.float32),
            jax.ShapeDtypeStruct((N, 1), jnp.float32),
        ],
    )(hists, x, W1)


def _tc_mid(aggp, ht1, dinv, b1, W2):
    return pl.pallas_call(
        _mid_body,
        grid=(GRID,),
        in_specs=[
            pl.BlockSpec((NC, BR, HID), lambda i: (0, i, 0)),
            pl.BlockSpec((BR, HID), lambda i: (i, 0)),
            pl.BlockSpec((BR, 1), lambda i: (i, 0)),
            pl.BlockSpec((1, HID), lambda i: (0, 0)),
            pl.BlockSpec((HID, HID), lambda i: (0, 0)),
        ],
        out_specs=pl.BlockSpec((BR, HID), lambda i: (i, 0)),
        out_shape=jax.ShapeDtypeStruct((N, HID), jnp.float32),
    )(aggp, ht1, dinv, b1, W2)


def _tc_pool(aggp, ht2, dinv, b2, batch2d, Wh, bh):
    return pl.pallas_call(
        _pool_body,
        grid=(GRID,),
        in_specs=[
            pl.BlockSpec((NC, BR, HID), lambda i: (0, i, 0)),
            pl.BlockSpec((BR, HID), lambda i: (i, 0)),
            pl.BlockSpec((BR, 1), lambda i: (i, 0)),
            pl.BlockSpec((1, HID), lambda i: (0, 0)),
            pl.BlockSpec((1, BR), lambda i: (0, i)),
            pl.BlockSpec((HID, OUT), lambda i: (0, 0)),
            pl.BlockSpec((1, OUT), lambda i: (0, 0)),
        ],
        out_specs=pl.BlockSpec((G, OUT), lambda i: (0, 0)),
        out_shape=jax.ShapeDtypeStruct((G, OUT), jnp.float32),
        scratch_shapes=[
            pltpu.VMEM((G, HID), jnp.float32),
            pltpu.VMEM((G, 1), jnp.float32),
        ],
    )(aggp, ht2, dinv, b2, batch2d, Wh, bh)


def kernel(x, edge_index, batch, W1, b1, W2, b2, Wh, bh):
    src = edge_index[0]
    dst = edge_index[1]
    zeros1 = jnp.zeros((N,), jnp.float32)
    zrows = jnp.zeros((RPT, HID), jnp.float32)
    b1r = b1.reshape(1, HID)
    b2r = b2.reshape(1, HID)
    bhr = bh.reshape(1, OUT)
    batch2d = batch.reshape(1, N)

    hists = _deg_kernel(dst, zeros1)
    ht1, dinv = _tc_mm1(hists, x, W1)
    agg1 = _agg_kernel(ht1, src, dst, zrows)
    ht2 = _tc_mid(agg1, ht1, dinv, b1r, W2)
    agg2 = _agg_kernel(ht2, src, dst, zrows)
    return _tc_pool(agg2, ht2, dinv, b2r, batch2d, Wh, bhr)


# speedup vs baseline: 13.7462x; 13.7462x over previous
import functools

import jax
import jax.numpy as jnp
from jax import lax
from jax.experimental import pallas as pl
from jax.experimental.pallas import tpu as pltpu
from jax.experimental.pallas import tpu_sc as plsc

N = 10000
D = 128
E = 320000
G = 16
HID = 128
OUT = 128

NC = 2
NS = 16
NW = NC * NS
EPT = E // NW
EPC = E // NC
K = 80
NCH = EPT // K
NP = 10240
RPT = NP // NS

_MESH = plsc.VectorSubcoreMesh(
    core_axis_name="c", subcore_axis_name="s", num_cores=NC, num_subcores=NS)


@functools.partial(
    pl.kernel,
    out_type=jax.ShapeDtypeStruct((NW, N), jnp.float32),
    mesh=_MESH,
    compiler_params=pltpu.CompilerParams(needs_layout_passes=False),
    scratch_types=[
        pltpu.VMEM((EPT,), jnp.int32),
        pltpu.VMEM((N,), jnp.float32),
    ],
)
def _deg_kernel(dst_hbm, zeros_hbm, out_hbm, idx_v, hist_v):
    c = lax.axis_index("c")
    s = lax.axis_index("s")
    wid = s * NC + c
    pltpu.sync_copy(zeros_hbm, hist_v)
    pltpu.sync_copy(dst_hbm.at[pl.ds(wid * EPT, EPT)], idx_v)
    ones = jnp.full((16,), 1.0, dtype=jnp.float32)

    @pl.loop(0, EPT // 16)
    def _(j):
        idx16 = idx_v[pl.ds(j * 16, 16)]
        plsc.addupdate_scatter(hist_v, [idx16], ones)

    pltpu.sync_copy(hist_v, out_hbm.at[wid])


@functools.partial(
    pl.kernel,
    out_type=jax.ShapeDtypeStruct((NC, NP, HID), jnp.float32),
    mesh=_MESH,
    compiler_params=pltpu.CompilerParams(needs_layout_passes=False),
    scratch_types=[
        pltpu.VMEM((K,), jnp.int32),
        pltpu.VMEM((1, K), jnp.int32),
        pltpu.VMEM((K, HID), jnp.float32),
        pltpu.VMEM_SHARED((NP, HID), jnp.float32),
        pltpu.SemaphoreType.DMA,
    ],
)
def _agg_kernel(ht_hbm, src_hbm, dst_hbm, zrows_hbm, out_hbm,
                sidx, didx, rows, acc, sem):
    c = lax.axis_index("c")
    s = lax.axis_index("s")
    base = c * EPC + s * EPT
    pltpu.sync_copy(zrows_hbm, acc.at[pl.ds(s * RPT, RPT)])
    plsc.subcore_barrier()

    @pl.loop(0, NCH)
    def _(i):
        eb = base + i * K
        pltpu.sync_copy(src_hbm.at[pl.ds(eb, K)], sidx)
        pltpu.sync_copy(dst_hbm.at[pl.ds(eb, K)], didx.at[0])
        pltpu.async_copy(ht_hbm.at[sidx], rows, sem).wait()
        pltpu.sync_copy(rows, acc.at[didx.at[0]], add=True)

    plsc.subcore_barrier()
    pltpu.sync_copy(acc.at[pl.ds(s * RPT, RPT)],
                    out_hbm.at[c, pl.ds(s * RPT, RPT)])


BR = 400
GRID = N // BR


def _mm1_body(hists_ref, x_ref, w1_ref, ht_ref, dinv_ref):
    deg = jnp.sum(hists_ref[...], axis=1) + 1.0
    dinv = lax.rsqrt(deg)
    h = jnp.dot(x_ref[...], w1_ref[...], preferred_element_type=jnp.float32)
    ht_ref[...] = h * dinv[:, None]
    dinv_ref[...] = dinv[:, None]


def _mid_body(aggp_ref, ht1_ref, dinv_ref, b1_ref, w2_ref, ht2_ref):
    dinv = dinv_ref[...]
    a = aggp_ref[...]
    m = dinv * (a[0] + a[1] + ht1_ref[...]) + b1_ref[...]
    h1r = jnp.maximum(m, 0.0)
    ht2_ref[...] = jnp.dot(
        h1r, w2_ref[...], preferred_element_type=jnp.float32) * dinv


def _pool_body(aggp_ref, ht2_ref, dinv_ref, b2_ref, batch_ref, wh_ref,
               bh_ref, out_ref, psum, pcnt):
    i = pl.program_id(0)

    @pl.when(i == 0)
    def _():
        psum[...] = jnp.zeros_like(psum)
        pcnt[...] = jnp.zeros_like(pcnt)

    a = aggp_ref[...]
    m = dinv_ref[...] * (a[0] + a[1] + ht2_ref[...]) + b2_ref[...]
    gids = lax.broadcasted_iota(jnp.int32, (BR, G), 1)
    oh = (gids == batch_ref[...]).astype(jnp.float32)
    psum[...] += lax.dot_general(
        oh, m, dimension_numbers=(((0,), (0,)), ((), ())),
        preferred_element_type=jnp.float32)
    pcnt[...] += jnp.sum(oh, axis=0)[:, None]

    @pl.when(i == GRID - 1)
    def _():
        pooled = psum[...] / jnp.maximum(pcnt[...], 1.0)
        out_ref[...] = jnp.dot(
            pooled, wh_ref[...], preferred_element_type=jnp.float32) + bh_ref[...]


def _tc_mm1(hists, x, W1):
    return pl.pallas_call(
        _mm1_body,
        grid=(GRID,),
        in_specs=[
            pl.BlockSpec((BR, NW), lambda i: (i, 0)),
            pl.BlockSpec((BR, D), lambda i: (i, 0)),
            pl.BlockSpec((D, HID), lambda i: (0, 0)),
        ],
        out_specs=[
            pl.BlockSpec((BR, HID), lambda i: (i, 0)),
            pl.BlockSpec((BR, 1), lambda i: (i, 0)),
        ],
        out_shape=[
            jax.ShapeDtypeStruct((N, HID), jnp.float32),
            jax.ShapeDtypeStruct((N, 1), jnp.float32),
        ],
    )(hists, x, W1)


def _tc_mid(aggp, ht1, dinv, b1, W2):
    return pl.pallas_call(
        _mid_body,
        grid=(GRID,),
        in_specs=[
            pl.BlockSpec((NC, BR, HID), lambda i: (0, i, 0)),
            pl.BlockSpec((BR, HID), lambda i: (i, 0)),
            pl.BlockSpec((BR, 1), lambda i: (i, 0)),
            pl.BlockSpec((1, HID), lambda i: (0, 0)),
            pl.BlockSpec((HID, HID), lambda i: (0, 0)),
        ],
        out_specs=pl.BlockSpec((BR, HID), lambda i: (i, 0)),
        out_shape=jax.ShapeDtypeStruct((N, HID), jnp.float32),
    )(aggp, ht1, dinv, b1, W2)


def _tc_pool(aggp, ht2, dinv, b2, batch2d, Wh, bh):
    return pl.pallas_call(
        _pool_body,
        grid=(GRID,),
        in_specs=[
            pl.BlockSpec((NC, BR, HID), lambda i: (0, i, 0)),
            pl.BlockSpec((BR, HID), lambda i: (i, 0)),
            pl.BlockSpec((BR, 1), lambda i: (i, 0)),
            pl.BlockSpec((1, HID), lambda i: (0, 0)),
            pl.BlockSpec((BR, 1), lambda i: (i, 0)),
            pl.BlockSpec((HID, OUT), lambda i: (0, 0)),
            pl.BlockSpec((1, OUT), lambda i: (0, 0)),
        ],
        out_specs=pl.BlockSpec((G, OUT), lambda i: (0, 0)),
        out_shape=jax.ShapeDtypeStruct((G, OUT), jnp.float32),
        scratch_shapes=[
            pltpu.VMEM((G, HID), jnp.float32),
            pltpu.VMEM((G, 1), jnp.float32),
        ],
    )(aggp, ht2, dinv, b2, batch2d, Wh, bh)


def kernel(x, edge_index, batch, W1, b1, W2, b2, Wh, bh):
    src = edge_index[0]
    dst = edge_index[1]
    zeros1 = jnp.zeros((N,), jnp.float32)
    zrows = jnp.zeros((RPT, HID), jnp.float32)
    b1r = b1.reshape(1, HID)
    b2r = b2.reshape(1, HID)
    bhr = bh.reshape(1, OUT)
    batch2d = batch.reshape(N, 1)

    hists = _deg_kernel(dst, zeros1)
    ht1, dinv = _tc_mm1(hists.T, x, W1)
    agg1 = _agg_kernel(ht1, src, dst, zrows)
    ht2 = _tc_mid(agg1, ht1, dinv, b1r, W2)
    agg2 = _agg_kernel(ht2, src, dst, zrows)
    return _tc_pool(agg2, ht2, dinv, b2r, batch2d, Wh, bhr)


# speedup vs baseline: 22.9017x; 1.6660x over previous
import functools

import jax
import jax.numpy as jnp
from jax import lax
from jax.experimental import pallas as pl
from jax.experimental.pallas import tpu as pltpu
from jax.experimental.pallas import tpu_sc as plsc

N = 10000
D = 128
E = 320000
G = 16
HID = 128
OUT = 128

NC = 2
NS = 16
NW = NC * NS
EPT = E // NW
EPC = E // NC
K = 80
NCH = EPT // K
CPB = 25
NBLK = NCH // CPB
NP = 10240
RPT = NP // NS

_MESH = plsc.VectorSubcoreMesh(
    core_axis_name="c", subcore_axis_name="s", num_cores=NC, num_subcores=NS)


@functools.partial(
    pl.kernel,
    out_type=jax.ShapeDtypeStruct((NW, N), jnp.float32),
    mesh=_MESH,
    compiler_params=pltpu.CompilerParams(needs_layout_passes=False),
    scratch_types=[
        pltpu.VMEM((EPT,), jnp.int32),
        pltpu.VMEM((N,), jnp.float32),
    ],
)
def _deg_kernel(dst_hbm, zeros_hbm, out_hbm, idx_v, hist_v):
    c = lax.axis_index("c")
    s = lax.axis_index("s")
    wid = s * NC + c
    pltpu.sync_copy(zeros_hbm, hist_v)
    pltpu.sync_copy(dst_hbm.at[pl.ds(wid * EPT, EPT)], idx_v)
    ones = jnp.full((16,), 1.0, dtype=jnp.float32)

    @pl.loop(0, EPT // 16)
    def _(j):
        idx16 = idx_v[pl.ds(j * 16, 16)]
        plsc.addupdate_scatter(hist_v, [idx16], ones)

    pltpu.sync_copy(hist_v, out_hbm.at[wid])


@functools.partial(
    pl.kernel,
    out_type=jax.ShapeDtypeStruct((NC, NP, HID), jnp.float32),
    mesh=_MESH,
    compiler_params=pltpu.CompilerParams(needs_layout_passes=False),
    scratch_types=[
        pltpu.VMEM((CPB, K), jnp.int32),
        pltpu.VMEM((CPB, K), jnp.int32),
        pltpu.VMEM((K, HID), jnp.float32),
        pltpu.VMEM((K, HID), jnp.float32),
        pltpu.VMEM_SHARED((NP, HID), jnp.float32),
        pltpu.SemaphoreType.DMA,
        pltpu.SemaphoreType.DMA,
    ],
)
def _agg_kernel(ht_hbm, src_hbm, dst_hbm, zrows_hbm, out_hbm,
                sidxb, didxb, rows0, rows1, acc, g0, g1):
    c = lax.axis_index("c")
    s = lax.axis_index("s")
    wid = c * NS + s
    pltpu.sync_copy(zrows_hbm, acc.at[pl.ds(s * RPT, RPT)])
    plsc.subcore_barrier()

    @pl.loop(0, NBLK)
    def _(blk):
        pltpu.sync_copy(src_hbm.at[wid, blk], sidxb)
        pltpu.sync_copy(dst_hbm.at[wid, blk], didxb)
        pltpu.async_copy(ht_hbm.at[sidxb.at[0]], rows0, g0)

        @pl.loop(0, CPB)
        def _(i):
            even = lax.rem(i, 2) == 0

            @pl.when(even)
            def _():
                pltpu.make_async_copy(ht_hbm.at[sidxb.at[0]], rows0, g0).wait()

                @pl.when(i + 1 < CPB)
                def _():
                    pltpu.async_copy(ht_hbm.at[sidxb.at[i + 1]], rows1, g1)

                pltpu.sync_copy(rows0, acc.at[didxb.at[i]], add=True)

            @pl.when(jnp.logical_not(even))
            def _():
                pltpu.make_async_copy(ht_hbm.at[sidxb.at[0]], rows1, g1).wait()

                @pl.when(i + 1 < CPB)
                def _():
                    pltpu.async_copy(ht_hbm.at[sidxb.at[i + 1]], rows0, g0)

                pltpu.sync_copy(rows1, acc.at[didxb.at[i]], add=True)

    plsc.subcore_barrier()
    pltpu.sync_copy(acc.at[pl.ds(s * RPT, RPT)],
                    out_hbm.at[c, pl.ds(s * RPT, RPT)])


BR = 400
GRID = N // BR


def _mm1_body(hists_ref, x_ref, w1_ref, ht_ref, dinv_ref):
    deg = jnp.sum(hists_ref[...], axis=1) + 1.0
    dinv = lax.rsqrt(deg)
    h = jnp.dot(x_ref[...], w1_ref[...], preferred_element_type=jnp.float32)
    ht_ref[...] = h * dinv[:, None]
    dinv_ref[...] = dinv[:, None]


def _mid_body(aggp_ref, ht1_ref, dinv_ref, b1_ref, w2_ref, ht2_ref):
    dinv = dinv_ref[...]
    a = aggp_ref[...]
    m = dinv * (a[0] + a[1] + ht1_ref[...]) + b1_ref[...]
    h1r = jnp.maximum(m, 0.0)
    ht2_ref[...] = jnp.dot(
        h1r, w2_ref[...], preferred_element_type=jnp.float32) * dinv


def _pool_body(aggp_ref, ht2_ref, dinv_ref, b2_ref, batch_ref, wh_ref,
               bh_ref, out_ref, psum, pcnt):
    i = pl.program_id(0)

    @pl.when(i == 0)
    def _():
        psum[...] = jnp.zeros_like(psum)
        pcnt[...] = jnp.zeros_like(pcnt)

    a = aggp_ref[...]
    m = dinv_ref[...] * (a[0] + a[1] + ht2_ref[...]) + b2_ref[...]
    gids = lax.broadcasted_iota(jnp.int32, (BR, G), 1)
    oh = (gids == batch_ref[...]).astype(jnp.float32)
    psum[...] += lax.dot_general(
        oh, m, dimension_numbers=(((0,), (0,)), ((), ())),
        preferred_element_type=jnp.float32)
    pcnt[...] += jnp.sum(oh, axis=0)[:, None]

    @pl.when(i == GRID - 1)
    def _():
        pooled = psum[...] / jnp.maximum(pcnt[...], 1.0)
        out_ref[...] = jnp.dot(
            pooled, wh_ref[...], preferred_element_type=jnp.float32) + bh_ref[...]


def _tc_mm1(hists, x, W1):
    return pl.pallas_call(
        _mm1_body,
        grid=(GRID,),
        in_specs=[
            pl.BlockSpec((BR, NW), lambda i: (i, 0)),
            pl.BlockSpec((BR, D), lambda i: (i, 0)),
            pl.BlockSpec((D, HID), lambda i: (0, 0)),
        ],
        out_specs=[
            pl.BlockSpec((BR, HID), lambda i: (i, 0)),
            pl.BlockSpec((BR, 1), lambda i: (i, 0)),
        ],
        out_shape=[
            jax.ShapeDtypeStruct((N, HID), jnp.float32),
            jax.ShapeDtypeStruct((N, 1), jnp.float32),
        ],
    )(hists, x, W1)


def _tc_mid(aggp, ht1, dinv, b1, W2):
    return pl.pallas_call(
        _mid_body,
        grid=(GRID,),
        in_specs=[
            pl.BlockSpec((NC, BR, HID), lambda i: (0, i, 0)),
            pl.BlockSpec((BR, HID), lambda i: (i, 0)),
            pl.BlockSpec((BR, 1), lambda i: (i, 0)),
            pl.BlockSpec((1, HID), lambda i: (0, 0)),
            pl.BlockSpec((HID, HID), lambda i: (0, 0)),
        ],
        out_specs=pl.BlockSpec((BR, HID), lambda i: (i, 0)),
        out_shape=jax.ShapeDtypeStruct((N, HID), jnp.float32),
    )(aggp, ht1, dinv, b1, W2)


def _tc_pool(aggp, ht2, dinv, b2, batch2d, Wh, bh):
    return pl.pallas_call(
        _pool_body,
        grid=(GRID,),
        in_specs=[
            pl.BlockSpec((NC, BR, HID), lambda i: (0, i, 0)),
            pl.BlockSpec((BR, HID), lambda i: (i, 0)),
            pl.BlockSpec((BR, 1), lambda i: (i, 0)),
            pl.BlockSpec((1, HID), lambda i: (0, 0)),
            pl.BlockSpec((BR, 1), lambda i: (i, 0)),
            pl.BlockSpec((HID, OUT), lambda i: (0, 0)),
            pl.BlockSpec((1, OUT), lambda i: (0, 0)),
        ],
        out_specs=pl.BlockSpec((G, OUT), lambda i: (0, 0)),
        out_shape=jax.ShapeDtypeStruct((G, OUT), jnp.float32),
        scratch_shapes=[
            pltpu.VMEM((G, HID), jnp.float32),
            pltpu.VMEM((G, 1), jnp.float32),
        ],
    )(aggp, ht2, dinv, b2, batch2d, Wh, bh)


def kernel(x, edge_index, batch, W1, b1, W2, b2, Wh, bh):
    src = edge_index[0]
    dst = edge_index[1]
    zeros1 = jnp.zeros((N,), jnp.float32)
    zrows = jnp.zeros((RPT, HID), jnp.float32)
    b1r = b1.reshape(1, HID)
    b2r = b2.reshape(1, HID)
    bhr = bh.reshape(1, OUT)
    batch2d = batch.reshape(N, 1)

    src3 = src.reshape(NW, NBLK, CPB, K)
    dst3 = dst.reshape(NW, NBLK, CPB, K)

    hists = _deg_kernel(dst, zeros1)
    ht1, dinv = _tc_mm1(hists.T, x, W1)
    agg1 = _agg_kernel(ht1, src3, dst3, zrows)
    ht2 = _tc_mid(agg1, ht1, dinv, b1r, W2)
    agg2 = _agg_kernel(ht2, src3, dst3, zrows)
    return _tc_pool(agg2, ht2, dinv, b2r, batch2d, Wh, bhr)


# speedup vs baseline: 24.5688x; 1.0728x over previous
import functools

import jax
import jax.numpy as jnp
from jax import lax
from jax.experimental import pallas as pl
from jax.experimental.pallas import tpu as pltpu
from jax.experimental.pallas import tpu_sc as plsc

N = 10000
D = 128
E = 320000
G = 16
HID = 128
OUT = 128

NC = 2
NS = 16
NW = NC * NS
EPT = E // NW
EPC = E // NC
K = 40
NCH = EPT // K
NBUF = 4
CPB = 50
NBLK = NCH // CPB
NP = 10240
RPT = NP // NS

_MESH = plsc.VectorSubcoreMesh(
    core_axis_name="c", subcore_axis_name="s", num_cores=NC, num_subcores=NS)


@functools.partial(
    pl.kernel,
    out_type=jax.ShapeDtypeStruct((NW, N), jnp.float32),
    mesh=_MESH,
    compiler_params=pltpu.CompilerParams(needs_layout_passes=False),
    scratch_types=[
        pltpu.VMEM((EPT,), jnp.int32),
        pltpu.VMEM((N,), jnp.float32),
    ],
)
def _deg_kernel(dst_hbm, zeros_hbm, out_hbm, idx_v, hist_v):
    c = lax.axis_index("c")
    s = lax.axis_index("s")
    wid = s * NC + c
    pltpu.sync_copy(zeros_hbm, hist_v)
    pltpu.sync_copy(dst_hbm.at[pl.ds(wid * EPT, EPT)], idx_v)
    ones = jnp.full((16,), 1.0, dtype=jnp.float32)

    @pl.loop(0, EPT // 16)
    def _(j):
        idx16 = idx_v[pl.ds(j * 16, 16)]
        plsc.addupdate_scatter(hist_v, [idx16], ones)

    pltpu.sync_copy(hist_v, out_hbm.at[wid])


@functools.partial(
    pl.kernel,
    out_type=jax.ShapeDtypeStruct((NC, NP, HID), jnp.float32),
    mesh=_MESH,
    compiler_params=pltpu.CompilerParams(needs_layout_passes=False),
    scratch_types=[
        pltpu.VMEM((CPB, K), jnp.int32),
        pltpu.VMEM((CPB, K), jnp.int32),
        [pltpu.VMEM((K, HID), jnp.float32)] * NBUF,
        [pltpu.SemaphoreType.DMA] * NBUF,
        [pltpu.SemaphoreType.DMA] * NBUF,
        pltpu.VMEM_SHARED((NP, HID), jnp.float32),
    ],
)
def _agg_kernel(ht_hbm, src_hbm, dst_hbm, zrows_hbm, out_hbm,
                sidxb, didxb, rows, gsem, ssem, acc):
    c = lax.axis_index("c")
    s = lax.axis_index("s")
    wid = c * NS + s
    pltpu.sync_copy(zrows_hbm, acc.at[pl.ds(s * RPT, RPT)])
    plsc.subcore_barrier()

    @pl.loop(0, NBLK)
    def _(blk):
        pltpu.sync_copy(src_hbm.at[wid, blk], sidxb)
        pltpu.sync_copy(dst_hbm.at[wid, blk], didxb)

        pltpu.async_copy(ht_hbm.at[sidxb.at[0]], rows[0], gsem[0])
        pltpu.async_copy(ht_hbm.at[sidxb.at[1]], rows[1], gsem[1])

        @pl.loop(0, CPB)
        def _(i):
            for b in range(NBUF):
                @pl.when(lax.rem(i, NBUF) == b)
                def _(b=b):
                    b2 = (b + 2) % NBUF
                    pltpu.make_async_copy(
                        ht_hbm.at[sidxb.at[0]], rows[b], gsem[b]).wait()
                    pltpu.async_copy(rows[b], acc.at[didxb.at[i]], ssem[b],
                                     add=True)

                    @pl.when(i + 2 < CPB)
                    def _():
                        @pl.when(i >= 2)
                        def _():
                            pltpu.make_async_copy(
                                rows[b2], acc.at[didxb.at[0]],
                                ssem[b2]).wait()

                        pltpu.async_copy(ht_hbm.at[sidxb.at[i + 2]],
                                         rows[b2], gsem[b2])

        for b in range(NBUF):
            pltpu.make_async_copy(rows[b], acc.at[didxb.at[0]],
                                  ssem[b]).wait()

    plsc.subcore_barrier()
    pltpu.sync_copy(acc.at[pl.ds(s * RPT, RPT)],
                    out_hbm.at[c, pl.ds(s * RPT, RPT)])


BR = 400
GRID = N // BR


def _mm1_body(hists_ref, x_ref, w1_ref, ht_ref, dinv_ref):
    deg = jnp.sum(hists_ref[...], axis=1) + 1.0
    dinv = lax.rsqrt(deg)
    h = jnp.dot(x_ref[...], w1_ref[...], preferred_element_type=jnp.float32)
    ht_ref[...] = h * dinv[:, None]
    dinv_ref[...] = dinv[:, None]


def _mid_body(aggp_ref, ht1_ref, dinv_ref, b1_ref, w2_ref, ht2_ref):
    dinv = dinv_ref[...]
    a = aggp_ref[...]
    m = dinv * (a[0] + a[1] + ht1_ref[...]) + b1_ref[...]
    h1r = jnp.maximum(m, 0.0)
    ht2_ref[...] = jnp.dot(
        h1r, w2_ref[...], preferred_element_type=jnp.float32) * dinv


def _pool_body(aggp_ref, ht2_ref, dinv_ref, b2_ref, batch_ref, wh_ref,
               bh_ref, out_ref, psum, pcnt):
    i = pl.program_id(0)

    @pl.when(i == 0)
    def _():
        psum[...] = jnp.zeros_like(psum)
        pcnt[...] = jnp.zeros_like(pcnt)

    a = aggp_ref[...]
    m = dinv_ref[...] * (a[0] + a[1] + ht2_ref[...]) + b2_ref[...]
    gids = lax.broadcasted_iota(jnp.int32, (BR, G), 1)
    oh = (gids == batch_ref[...]).astype(jnp.float32)
    psum[...] += lax.dot_general(
        oh, m, dimension_numbers=(((0,), (0,)), ((), ())),
        preferred_element_type=jnp.float32)
    pcnt[...] += jnp.sum(oh, axis=0)[:, None]

    @pl.when(i == GRID - 1)
    def _():
        pooled = psum[...] / jnp.maximum(pcnt[...], 1.0)
        out_ref[...] = jnp.dot(
            pooled, wh_ref[...], preferred_element_type=jnp.float32) + bh_ref[...]


def _tc_mm1(hists, x, W1):
    return pl.pallas_call(
        _mm1_body,
        grid=(GRID,),
        in_specs=[
            pl.BlockSpec((BR, NW), lambda i: (i, 0)),
            pl.BlockSpec((BR, D), lambda i: (i, 0)),
            pl.BlockSpec((D, HID), lambda i: (0, 0)),
        ],
        out_specs=[
            pl.BlockSpec((BR, HID), lambda i: (i, 0)),
            pl.BlockSpec((BR, 1), lambda i: (i, 0)),
        ],
        out_shape=[
            jax.ShapeDtypeStruct((N, HID), jnp.float32),
            jax.ShapeDtypeStruct((N, 1), jnp.float32),
        ],
    )(hists, x, W1)


def _tc_mid(aggp, ht1, dinv, b1, W2):
    return pl.pallas_call(
        _mid_body,
        grid=(GRID,),
        in_specs=[
            pl.BlockSpec((NC, BR, HID), lambda i: (0, i, 0)),
            pl.BlockSpec((BR, HID), lambda i: (i, 0)),
            pl.BlockSpec((BR, 1), lambda i: (i, 0)),
            pl.BlockSpec((1, HID), lambda i: (0, 0)),
            pl.BlockSpec((HID, HID), lambda i: (0, 0)),
        ],
        out_specs=pl.BlockSpec((BR, HID), lambda i: (i, 0)),
        out_shape=jax.ShapeDtypeStruct((N, HID), jnp.float32),
    )(aggp, ht1, dinv, b1, W2)


def _tc_pool(aggp, ht2, dinv, b2, batch2d, Wh, bh):
    return pl.pallas_call(
        _pool_body,
        grid=(GRID,),
        in_specs=[
            pl.BlockSpec((NC, BR, HID), lambda i: (0, i, 0)),
            pl.BlockSpec((BR, HID), lambda i: (i, 0)),
            pl.BlockSpec((BR, 1), lambda i: (i, 0)),
            pl.BlockSpec((1, HID), lambda i: (0, 0)),
            pl.BlockSpec((BR, 1), lambda i: (i, 0)),
            pl.BlockSpec((HID, OUT), lambda i: (0, 0)),
            pl.BlockSpec((1, OUT), lambda i: (0, 0)),
        ],
        out_specs=pl.BlockSpec((G, OUT), lambda i: (0, 0)),
        out_shape=jax.ShapeDtypeStruct((G, OUT), jnp.float32),
        scratch_shapes=[
            pltpu.VMEM((G, HID), jnp.float32),
            pltpu.VMEM((G, 1), jnp.float32),
        ],
    )(aggp, ht2, dinv, b2, batch2d, Wh, bh)


def kernel(x, edge_index, batch, W1, b1, W2, b2, Wh, bh):
    src = edge_index[0]
    dst = edge_index[1]
    zeros1 = jnp.zeros((N,), jnp.float32)
    zrows = jnp.zeros((RPT, HID), jnp.float32)
    b1r = b1.reshape(1, HID)
    b2r = b2.reshape(1, HID)
    bhr = bh.reshape(1, OUT)
    batch2d = batch.reshape(N, 1)

    src3 = src.reshape(NW, NBLK, CPB, K)
    dst3 = dst.reshape(NW, NBLK, CPB, K)

    hists = _deg_kernel(dst, zeros1)
    ht1, dinv = _tc_mm1(hists.T, x, W1)
    agg1 = _agg_kernel(ht1, src3, dst3, zrows)
    ht2 = _tc_mid(agg1, ht1, dinv, b1r, W2)
    agg2 = _agg_kernel(ht2, src3, dst3, zrows)
    return _tc_pool(agg2, ht2, dinv, b2r, batch2d, Wh, bhr)


# speedup vs baseline: 28.9034x; 1.1764x over previous
import functools

import jax
import jax.numpy as jnp
from jax import lax
from jax.experimental import pallas as pl
from jax.experimental.pallas import tpu as pltpu
from jax.experimental.pallas import tpu_sc as plsc

N = 10000
D = 128
E = 320000
G = 16
HID = 128
OUT = 128

NC = 2
NS = 16
NW = NC * NS
EPT = E // NW
EPC = E // NC
K = 40
NCH = EPT // K
NBUF = 4
CPB = 50
NBLK = NCH // CPB
NP = 10240
RPT = NP // NS

_MESH = plsc.VectorSubcoreMesh(
    core_axis_name="c", subcore_axis_name="s", num_cores=NC, num_subcores=NS)


@functools.partial(
    pl.kernel,
    out_type=jax.ShapeDtypeStruct((NW, N), jnp.float32),
    mesh=_MESH,
    compiler_params=pltpu.CompilerParams(needs_layout_passes=False),
    scratch_types=[
        pltpu.VMEM((EPT,), jnp.int32),
        pltpu.VMEM((N,), jnp.float32),
    ],
)
def _deg_kernel(dst_hbm, zeros_hbm, out_hbm, idx_v, hist_v):
    c = lax.axis_index("c")
    s = lax.axis_index("s")
    wid = s * NC + c
    pltpu.sync_copy(zeros_hbm, hist_v)
    pltpu.sync_copy(dst_hbm.at[pl.ds(wid * EPT, EPT)], idx_v)
    ones = jnp.full((16,), 1.0, dtype=jnp.float32)

    @pl.loop(0, EPT // 16)
    def _(j):
        idx16 = idx_v[pl.ds(j * 16, 16)]
        plsc.addupdate_scatter(hist_v, [idx16], ones)

    pltpu.sync_copy(hist_v, out_hbm.at[wid])


@functools.partial(
    pl.kernel,
    out_type=jax.ShapeDtypeStruct((NC, NP, HID), jnp.float32),
    mesh=_MESH,
    compiler_params=pltpu.CompilerParams(needs_layout_passes=False),
    scratch_types=[
        pltpu.VMEM((CPB, K), jnp.int32),
        pltpu.VMEM((CPB, K), jnp.int32),
        [pltpu.VMEM((K, HID), jnp.float32)] * NBUF,
        [pltpu.SemaphoreType.DMA] * NBUF,
        [pltpu.SemaphoreType.DMA] * NBUF,
        pltpu.VMEM_SHARED((NP, HID), jnp.float32),
    ],
)
def _agg_kernel(ht_hbm, src_hbm, dst_hbm, zrows_hbm, out_hbm,
                sidxb, didxb, rows, gsem, ssem, acc):
    c = lax.axis_index("c")
    s = lax.axis_index("s")
    wid = c * NS + s
    pltpu.sync_copy(zrows_hbm, acc.at[pl.ds(s * RPT, RPT)])
    plsc.subcore_barrier()

    @pl.loop(0, NBLK)
    def _(blk):
        pltpu.sync_copy(src_hbm.at[wid, blk], sidxb)
        pltpu.sync_copy(dst_hbm.at[wid, blk], didxb)

        pltpu.async_copy(ht_hbm.at[sidxb.at[0]], rows[0], gsem[0])
        pltpu.async_copy(ht_hbm.at[sidxb.at[1]], rows[1], gsem[1])
        pltpu.async_copy(ht_hbm.at[sidxb.at[2]], rows[2], gsem[2])

        @pl.loop(0, CPB)
        def _(i):
            for b in range(NBUF):
                @pl.when(lax.rem(i, NBUF) == b)
                def _(b=b):
                    b3 = (b + 3) % NBUF
                    pltpu.make_async_copy(
                        ht_hbm.at[sidxb.at[0]], rows[b], gsem[b]).wait()
                    pltpu.async_copy(rows[b], acc.at[didxb.at[i]], ssem[b],
                                     add=True)

                    @pl.when(i + 3 < CPB)
                    def _():
                        @pl.when(i >= 1)
                        def _():
                            pltpu.make_async_copy(
                                rows[b3], acc.at[didxb.at[0]],
                                ssem[b3]).wait()

                        pltpu.async_copy(ht_hbm.at[sidxb.at[i + 3]],
                                         rows[b3], gsem[b3])

        for b in range(NBUF):
            pltpu.make_async_copy(rows[b], acc.at[didxb.at[0]],
                                  ssem[b]).wait()

    plsc.subcore_barrier()
    pltpu.sync_copy(acc.at[pl.ds(s * RPT, RPT)],
                    out_hbm.at[c, pl.ds(s * RPT, RPT)])


BR = 400
GRID = N // BR


def _mm1_body(hists_ref, x_ref, w1_ref, ht_ref, dinv_ref):
    deg = jnp.sum(hists_ref[...], axis=1) + 1.0
    dinv = lax.rsqrt(deg)
    h = jnp.dot(x_ref[...], w1_ref[...], preferred_element_type=jnp.float32)
    ht_ref[...] = h * dinv[:, None]
    dinv_ref[...] = dinv[:, None]


def _mid_body(aggp_ref, ht1_ref, dinv_ref, b1_ref, w2_ref, ht2_ref):
    dinv = dinv_ref[...]
    a = aggp_ref[...]
    m = dinv * (a[0] + a[1] + ht1_ref[...]) + b1_ref[...]
    h1r = jnp.maximum(m, 0.0)
    ht2_ref[...] = jnp.dot(
        h1r, w2_ref[...], preferred_element_type=jnp.float32) * dinv


def _pool_body(aggp_ref, ht2_ref, dinv_ref, b2_ref, batch_ref, wh_ref,
               bh_ref, out_ref, psum, pcnt):
    i = pl.program_id(0)

    @pl.when(i == 0)
    def _():
        psum[...] = jnp.zeros_like(psum)
        pcnt[...] = jnp.zeros_like(pcnt)

    a = aggp_ref[...]
    m = dinv_ref[...] * (a[0] + a[1] + ht2_ref[...]) + b2_ref[...]
    gids = lax.broadcasted_iota(jnp.int32, (BR, G), 1)
    oh = (gids == batch_ref[...]).astype(jnp.float32)
    psum[...] += lax.dot_general(
        oh, m, dimension_numbers=(((0,), (0,)), ((), ())),
        preferred_element_type=jnp.float32)
    pcnt[...] += jnp.sum(oh, axis=0)[:, None]

    @pl.when(i == GRID - 1)
    def _():
        pooled = psum[...] / jnp.maximum(pcnt[...], 1.0)
        out_ref[...] = jnp.dot(
            pooled, wh_ref[...], preferred_element_type=jnp.float32) + bh_ref[...]


def _tc_mm1(hists, x, W1):
    return pl.pallas_call(
        _mm1_body,
        grid=(GRID,),
        in_specs=[
            pl.BlockSpec((BR, NW), lambda i: (i, 0)),
            pl.BlockSpec((BR, D), lambda i: (i, 0)),
            pl.BlockSpec((D, HID), lambda i: (0, 0)),
        ],
        out_specs=[
            pl.BlockSpec((BR, HID), lambda i: (i, 0)),
            pl.BlockSpec((BR, 1), lambda i: (i, 0)),
        ],
        out_shape=[
            jax.ShapeDtypeStruct((N, HID), jnp.float32),
            jax.ShapeDtypeStruct((N, 1), jnp.float32),
        ],
    )(hists, x, W1)


def _tc_mid(aggp, ht1, dinv, b1, W2):
    return pl.pallas_call(
        _mid_body,
        grid=(GRID,),
        in_specs=[
            pl.BlockSpec((NC, BR, HID), lambda i: (0, i, 0)),
            pl.BlockSpec((BR, HID), lambda i: (i, 0)),
            pl.BlockSpec((BR, 1), lambda i: (i, 0)),
            pl.BlockSpec((1, HID), lambda i: (0, 0)),
            pl.BlockSpec((HID, HID), lambda i: (0, 0)),
        ],
        out_specs=pl.BlockSpec((BR, HID), lambda i: (i, 0)),
        out_shape=jax.ShapeDtypeStruct((N, HID), jnp.float32),
    )(aggp, ht1, dinv, b1, W2)


def _tc_pool(aggp, ht2, dinv, b2, batch2d, Wh, bh):
    return pl.pallas_call(
        _pool_body,
        grid=(GRID,),
        in_specs=[
            pl.BlockSpec((NC, BR, HID), lambda i: (0, i, 0)),
            pl.BlockSpec((BR, HID), lambda i: (i, 0)),
            pl.BlockSpec((BR, 1), lambda i: (i, 0)),
            pl.BlockSpec((1, HID), lambda i: (0, 0)),
            pl.BlockSpec((BR, 1), lambda i: (i, 0)),
            pl.BlockSpec((HID, OUT), lambda i: (0, 0)),
            pl.BlockSpec((1, OUT), lambda i: (0, 0)),
        ],
        out_specs=pl.BlockSpec((G, OUT), lambda i: (0, 0)),
        out_shape=jax.ShapeDtypeStruct((G, OUT), jnp.float32),
        scratch_shapes=[
            pltpu.VMEM((G, HID), jnp.float32),
            pltpu.VMEM((G, 1), jnp.float32),
        ],
    )(aggp, ht2, dinv, b2, batch2d, Wh, bh)


def kernel(x, edge_index, batch, W1, b1, W2, b2, Wh, bh):
    src = edge_index[0]
    dst = edge_index[1]
    zeros1 = jnp.zeros((N,), jnp.float32)
    zrows = jnp.zeros((RPT, HID), jnp.float32)
    b1r = b1.reshape(1, HID)
    b2r = b2.reshape(1, HID)
    bhr = bh.reshape(1, OUT)
    batch2d = batch.reshape(N, 1)

    src3 = src.reshape(NW, NBLK, CPB, K)
    dst3 = dst.reshape(NW, NBLK, CPB, K)

    hists = _deg_kernel(dst, zeros1)
    ht1, dinv = _tc_mm1(hists.T, x, W1)
    agg1 = _agg_kernel(ht1, src3, dst3, zrows)
    ht2 = _tc_mid(agg1, ht1, dinv, b1r, W2)
    agg2 = _agg_kernel(ht2, src3, dst3, zrows)
    return _tc_pool(agg2, ht2, dinv, b2r, batch2d, Wh, bhr)


# speedup vs baseline: 31.6971x; 1.0967x over previous
import functools

import jax
import jax.numpy as jnp
from jax import lax
from jax.experimental import pallas as pl
from jax.experimental.pallas import tpu as pltpu
from jax.experimental.pallas import tpu_sc as plsc

N = 10000
D = 128
E = 320000
G = 16
HID = 128
OUT = 128

NC = 2
NS = 16
NW = NC * NS
EPT = E // NW
EPC = E // NC
K = 40
NCH = EPT // K
NBUF = 4
CPB = 50
NBLK = NCH // CPB
NP = 10240
RPT = NP // NS

_MESH = plsc.VectorSubcoreMesh(
    core_axis_name="c", subcore_axis_name="s", num_cores=NC, num_subcores=NS)


@functools.partial(
    pl.kernel,
    out_type=jax.ShapeDtypeStruct((NW, N), jnp.float32),
    mesh=_MESH,
    compiler_params=pltpu.CompilerParams(needs_layout_passes=False),
    scratch_types=[
        pltpu.VMEM((EPT,), jnp.int32),
        pltpu.VMEM((N,), jnp.float32),
    ],
)
def _deg_kernel(dst_hbm, zeros_hbm, out_hbm, idx_v, hist_v):
    c = lax.axis_index("c")
    s = lax.axis_index("s")
    wid = s * NC + c
    pltpu.sync_copy(zeros_hbm, hist_v)
    pltpu.sync_copy(dst_hbm.at[pl.ds(wid * EPT, EPT)], idx_v)
    ones = jnp.full((16,), 1.0, dtype=jnp.float32)

    @pl.loop(0, EPT // 16)
    def _(j):
        idx16 = idx_v[pl.ds(j * 16, 16)]
        plsc.addupdate_scatter(hist_v, [idx16], ones)

    pltpu.sync_copy(hist_v, out_hbm.at[wid])


@functools.partial(
    pl.kernel,
    out_type=jax.ShapeDtypeStruct((NC, NP, HID), jnp.float32),
    mesh=_MESH,
    compiler_params=pltpu.CompilerParams(needs_layout_passes=False),
    scratch_types=[
        pltpu.VMEM((CPB, K), jnp.int32),
        pltpu.VMEM((CPB, K), jnp.int32),
        [pltpu.VMEM((K, HID), jnp.float32)] * NBUF,
        [pltpu.SemaphoreType.DMA] * NBUF,
        [pltpu.SemaphoreType.DMA] * NBUF,
        pltpu.VMEM_SHARED((NP, HID), jnp.float32),
    ],
)
def _agg_kernel(ht_hbm, src_hbm, dst_hbm, zrows_hbm, out_hbm,
                sidxb, didxb, rows, gsem, ssem, acc):
    c = lax.axis_index("c")
    s = lax.axis_index("s")
    wid = c * NS + s
    pltpu.sync_copy(zrows_hbm, acc.at[pl.ds(s * RPT, RPT)])
    plsc.subcore_barrier()

    @pl.loop(0, NBLK)
    def _(blk):
        pltpu.sync_copy(src_hbm.at[wid, blk], sidxb)
        pltpu.sync_copy(dst_hbm.at[wid, blk], didxb)

        pltpu.async_copy(ht_hbm.at[sidxb.at[0]], rows[0], gsem[0])
        pltpu.async_copy(ht_hbm.at[sidxb.at[1]], rows[1], gsem[1])
        pltpu.async_copy(ht_hbm.at[sidxb.at[2]], rows[2], gsem[2])

        @pl.loop(0, CPB)
        def _(i):
            for b in range(NBUF):
                @pl.when(lax.rem(i, NBUF) == b)
                def _(b=b):
                    b3 = (b + 3) % NBUF
                    pltpu.make_async_copy(
                        ht_hbm.at[sidxb.at[0]], rows[b], gsem[b]).wait()
                    pltpu.async_copy(rows[b], acc.at[didxb.at[i]], ssem[b],
                                     add=True)

                    @pl.when(i + 3 < CPB)
                    def _():
                        @pl.when(i >= 1)
                        def _():
                            pltpu.make_async_copy(
                                rows[b3], acc.at[didxb.at[0]],
                                ssem[b3]).wait()

                        pltpu.async_copy(ht_hbm.at[sidxb.at[i + 3]],
                                         rows[b3], gsem[b3])

        for b in range(NBUF):
            pltpu.make_async_copy(rows[b], acc.at[didxb.at[0]],
                                  ssem[b]).wait()

    plsc.subcore_barrier()
    pltpu.sync_copy(acc.at[pl.ds(s * RPT, RPT)],
                    out_hbm.at[c, pl.ds(s * RPT, RPT)])


BR = 2000
GRID = N // BR


def _mm1_body(hists_ref, x_ref, w1_ref, ht_ref, dinv_ref):
    deg = jnp.sum(hists_ref[...], axis=1) + 1.0
    dinv = lax.rsqrt(deg)
    h = jnp.dot(x_ref[...], w1_ref[...], preferred_element_type=jnp.float32)
    ht_ref[...] = h * dinv[:, None]
    dinv_ref[...] = dinv[:, None]


def _mid_body(aggp_ref, ht1_ref, dinv_ref, b1_ref, w2_ref, ht2_ref):
    dinv = dinv_ref[...]
    a = aggp_ref[...]
    m = dinv * (a[0] + a[1] + ht1_ref[...]) + b1_ref[...]
    h1r = jnp.maximum(m, 0.0)
    ht2_ref[...] = jnp.dot(
        h1r, w2_ref[...], preferred_element_type=jnp.float32) * dinv


def _pool_body(aggp_ref, ht2_ref, dinv_ref, b2_ref, batch_ref, wh_ref,
               bh_ref, out_ref, psum, pcnt):
    i = pl.program_id(0)

    @pl.when(i == 0)
    def _():
        psum[...] = jnp.zeros_like(psum)
        pcnt[...] = jnp.zeros_like(pcnt)

    a = aggp_ref[...]
    m = dinv_ref[...] * (a[0] + a[1] + ht2_ref[...]) + b2_ref[...]
    gids = lax.broadcasted_iota(jnp.int32, (BR, G), 1)
    oh = (gids == batch_ref[...]).astype(jnp.float32)
    psum[...] += lax.dot_general(
        oh, m, dimension_numbers=(((0,), (0,)), ((), ())),
        preferred_element_type=jnp.float32)
    pcnt[...] += jnp.sum(oh, axis=0)[:, None]

    @pl.when(i == GRID - 1)
    def _():
        pooled = psum[...] / jnp.maximum(pcnt[...], 1.0)
        out_ref[...] = jnp.dot(
            pooled, wh_ref[...], preferred_element_type=jnp.float32) + bh_ref[...]


def _tc_mm1(hists, x, W1):
    return pl.pallas_call(
        _mm1_body,
        grid=(GRID,),
        in_specs=[
            pl.BlockSpec((BR, NW), lambda i: (i, 0)),
            pl.BlockSpec((BR, D), lambda i: (i, 0)),
            pl.BlockSpec((D, HID), lambda i: (0, 0)),
        ],
        out_specs=[
            pl.BlockSpec((BR, HID), lambda i: (i, 0)),
            pl.BlockSpec((BR, 1), lambda i: (i, 0)),
        ],
        out_shape=[
            jax.ShapeDtypeStruct((N, HID), jnp.float32),
            jax.ShapeDtypeStruct((N, 1), jnp.float32),
        ],
    )(hists, x, W1)


def _tc_mid(aggp, ht1, dinv, b1, W2):
    return pl.pallas_call(
        _mid_body,
        grid=(GRID,),
        in_specs=[
            pl.BlockSpec((NC, BR, HID), lambda i: (0, i, 0)),
            pl.BlockSpec((BR, HID), lambda i: (i, 0)),
            pl.BlockSpec((BR, 1), lambda i: (i, 0)),
            pl.BlockSpec((1, HID), lambda i: (0, 0)),
            pl.BlockSpec((HID, HID), lambda i: (0, 0)),
        ],
        out_specs=pl.BlockSpec((BR, HID), lambda i: (i, 0)),
        out_shape=jax.ShapeDtypeStruct((N, HID), jnp.float32),
    )(aggp, ht1, dinv, b1, W2)


def _tc_pool(aggp, ht2, dinv, b2, batch2d, Wh, bh):
    return pl.pallas_call(
        _pool_body,
        grid=(GRID,),
        in_specs=[
            pl.BlockSpec((NC, BR, HID), lambda i: (0, i, 0)),
            pl.BlockSpec((BR, HID), lambda i: (i, 0)),
            pl.BlockSpec((BR, 1), lambda i: (i, 0)),
            pl.BlockSpec((1, HID), lambda i: (0, 0)),
            pl.BlockSpec((BR, 1), lambda i: (i, 0)),
            pl.BlockSpec((HID, OUT), lambda i: (0, 0)),
            pl.BlockSpec((1, OUT), lambda i: (0, 0)),
        ],
        out_specs=pl.BlockSpec((G, OUT), lambda i: (0, 0)),
        out_shape=jax.ShapeDtypeStruct((G, OUT), jnp.float32),
        scratch_shapes=[
            pltpu.VMEM((G, HID), jnp.float32),
            pltpu.VMEM((G, 1), jnp.float32),
        ],
    )(aggp, ht2, dinv, b2, batch2d, Wh, bh)


def kernel(x, edge_index, batch, W1, b1, W2, b2, Wh, bh):
    src = edge_index[0]
    dst = edge_index[1]
    zeros1 = jnp.zeros((N,), jnp.float32)
    zrows = jnp.zeros((RPT, HID), jnp.float32)
    b1r = b1.reshape(1, HID)
    b2r = b2.reshape(1, HID)
    bhr = bh.reshape(1, OUT)
    batch2d = batch.reshape(N, 1)

    src3 = src.reshape(NW, NBLK, CPB, K)
    dst3 = dst.reshape(NW, NBLK, CPB, K)

    hists = _deg_kernel(dst, zeros1)
    ht1, dinv = _tc_mm1(hists.T, x, W1)
    agg1 = _agg_kernel(ht1, src3, dst3, zrows)
    ht2 = _tc_mid(agg1, ht1, dinv, b1r, W2)
    agg2 = _agg_kernel(ht2, src3, dst3, zrows)
    return _tc_pool(agg2, ht2, dinv, b2r, batch2d, Wh, bhr)


# speedup vs baseline: 32.0209x; 1.0102x over previous
import functools

import jax
import jax.numpy as jnp
from jax import lax
from jax.experimental import pallas as pl
from jax.experimental.pallas import tpu as pltpu
from jax.experimental.pallas import tpu_sc as plsc

N = 10000
D = 128
E = 320000
G = 16
HID = 128
OUT = 128

NC = 2
NS = 16
NW = NC * NS
EPT = E // NW
EPC = E // NC
K = 40
NCH = EPT // K
NBUF = 4
CPB = 50
NBLK = NCH // CPB
NP = 10240
RPT = NP // NS

_MESH = plsc.VectorSubcoreMesh(
    core_axis_name="c", subcore_axis_name="s", num_cores=NC, num_subcores=NS)


@functools.partial(
    pl.kernel,
    out_type=jax.ShapeDtypeStruct((NW, N), jnp.float32),
    mesh=_MESH,
    compiler_params=pltpu.CompilerParams(needs_layout_passes=False),
    scratch_types=[
        pltpu.VMEM((EPT,), jnp.int32),
        pltpu.VMEM((N,), jnp.float32),
    ],
)
def _deg_kernel(dst_hbm, zeros_hbm, out_hbm, idx_v, hist_v):
    c = lax.axis_index("c")
    s = lax.axis_index("s")
    wid = s * NC + c
    pltpu.sync_copy(zeros_hbm, hist_v)
    pltpu.sync_copy(dst_hbm.at[pl.ds(wid * EPT, EPT)], idx_v)
    ones = jnp.full((16,), 1.0, dtype=jnp.float32)

    @pl.loop(0, EPT // 16, unroll=8)
    def _(j):
        idx16 = idx_v[pl.ds(j * 16, 16)]
        plsc.addupdate_scatter(hist_v, [idx16], ones)

    pltpu.sync_copy(hist_v, out_hbm.at[wid])


@functools.partial(
    pl.kernel,
    out_type=jax.ShapeDtypeStruct((NC, NP, HID), jnp.float32),
    mesh=_MESH,
    compiler_params=pltpu.CompilerParams(needs_layout_passes=False),
    scratch_types=[
        pltpu.VMEM((CPB, K), jnp.int32),
        pltpu.VMEM((CPB, K), jnp.int32),
        [pltpu.VMEM((K, HID), jnp.float32)] * NBUF,
        [pltpu.SemaphoreType.DMA] * NBUF,
        [pltpu.SemaphoreType.DMA] * NBUF,
        pltpu.VMEM_SHARED((NP, HID), jnp.float32),
    ],
)
def _agg_kernel(ht_hbm, src_hbm, dst_hbm, zrows_hbm, out_hbm,
                sidxb, didxb, rows, gsem, ssem, acc):
    c = lax.axis_index("c")
    s = lax.axis_index("s")
    wid = c * NS + s
    pltpu.sync_copy(zrows_hbm, acc.at[pl.ds(s * RPT, RPT)])
    plsc.subcore_barrier()

    @pl.loop(0, NBLK)
    def _(blk):
        pltpu.sync_copy(src_hbm.at[wid, blk], sidxb)
        pltpu.sync_copy(dst_hbm.at[wid, blk], didxb)

        pltpu.async_copy(ht_hbm.at[sidxb.at[0]], rows[0], gsem[0])
        pltpu.async_copy(ht_hbm.at[sidxb.at[1]], rows[1], gsem[1])
        pltpu.async_copy(ht_hbm.at[sidxb.at[2]], rows[2], gsem[2])

        @pl.loop(0, CPB)
        def _(i):
            for b in range(NBUF):
                @pl.when(lax.rem(i, NBUF) == b)
                def _(b=b):
                    b3 = (b + 3) % NBUF
                    pltpu.make_async_copy(
                        ht_hbm.at[sidxb.at[0]], rows[b], gsem[b]).wait()
                    pltpu.async_copy(rows[b], acc.at[didxb.at[i]], ssem[b],
                                     add=True)

                    @pl.when(i + 3 < CPB)
                    def _():
                        @pl.when(i >= 1)
                        def _():
                            pltpu.make_async_copy(
                                rows[b3], acc.at[didxb.at[0]],
                                ssem[b3]).wait()

                        pltpu.async_copy(ht_hbm.at[sidxb.at[i + 3]],
                                         rows[b3], gsem[b3])

        for b in range(NBUF):
            pltpu.make_async_copy(rows[b], acc.at[didxb.at[0]],
                                  ssem[b]).wait()

    plsc.subcore_barrier()
    pltpu.sync_copy(acc.at[pl.ds(s * RPT, RPT)],
                    out_hbm.at[c, pl.ds(s * RPT, RPT)])


BR = 5000
GRID = N // BR


def _mm1_body(hists_ref, x_ref, w1_ref, ht_ref, dinv_ref):
    deg = jnp.sum(hists_ref[...], axis=1) + 1.0
    dinv = lax.rsqrt(deg)
    h = jnp.dot(x_ref[...], w1_ref[...], preferred_element_type=jnp.float32)
    ht_ref[...] = h * dinv[:, None]
    dinv_ref[...] = dinv[:, None]


def _mid_body(aggp_ref, ht1_ref, dinv_ref, b1_ref, w2_ref, ht2_ref):
    dinv = dinv_ref[...]
    a = aggp_ref[...]
    m = dinv * (a[0] + a[1] + ht1_ref[...]) + b1_ref[...]
    h1r = jnp.maximum(m, 0.0)
    ht2_ref[...] = jnp.dot(
        h1r, w2_ref[...], preferred_element_type=jnp.float32) * dinv


def _pool_body(aggp_ref, ht2_ref, dinv_ref, b2_ref, batch_ref, wh_ref,
               bh_ref, out_ref, psum, pcnt):
    i = pl.program_id(0)

    @pl.when(i == 0)
    def _():
        psum[...] = jnp.zeros_like(psum)
        pcnt[...] = jnp.zeros_like(pcnt)

    a = aggp_ref[...]
    m = dinv_ref[...] * (a[0] + a[1] + ht2_ref[...]) + b2_ref[...]
    gids = lax.broadcasted_iota(jnp.int32, (BR, G), 1)
    oh = (gids == batch_ref[...]).astype(jnp.float32)
    psum[...] += lax.dot_general(
        oh, m, dimension_numbers=(((0,), (0,)), ((), ())),
        preferred_element_type=jnp.float32)
    pcnt[...] += jnp.sum(oh, axis=0)[:, None]

    @pl.when(i == GRID - 1)
    def _():
        pooled = psum[...] / jnp.maximum(pcnt[...], 1.0)
        out_ref[...] = jnp.dot(
            pooled, wh_ref[...], preferred_element_type=jnp.float32) + bh_ref[...]


def _tc_mm1(hists, x, W1):
    return pl.pallas_call(
        _mm1_body,
        grid=(GRID,),
        in_specs=[
            pl.BlockSpec((BR, NW), lambda i: (i, 0)),
            pl.BlockSpec((BR, D), lambda i: (i, 0)),
            pl.BlockSpec((D, HID), lambda i: (0, 0)),
        ],
        out_specs=[
            pl.BlockSpec((BR, HID), lambda i: (i, 0)),
            pl.BlockSpec((BR, 1), lambda i: (i, 0)),
        ],
        out_shape=[
            jax.ShapeDtypeStruct((N, HID), jnp.float32),
            jax.ShapeDtypeStruct((N, 1), jnp.float32),
        ],
    )(hists, x, W1)


def _tc_mid(aggp, ht1, dinv, b1, W2):
    return pl.pallas_call(
        _mid_body,
        grid=(GRID,),
        in_specs=[
            pl.BlockSpec((NC, BR, HID), lambda i: (0, i, 0)),
            pl.BlockSpec((BR, HID), lambda i: (i, 0)),
            pl.BlockSpec((BR, 1), lambda i: (i, 0)),
            pl.BlockSpec((1, HID), lambda i: (0, 0)),
            pl.BlockSpec((HID, HID), lambda i: (0, 0)),
        ],
        out_specs=pl.BlockSpec((BR, HID), lambda i: (i, 0)),
        out_shape=jax.ShapeDtypeStruct((N, HID), jnp.float32),
    )(aggp, ht1, dinv, b1, W2)


def _tc_pool(aggp, ht2, dinv, b2, batch2d, Wh, bh):
    return pl.pallas_call(
        _pool_body,
        grid=(GRID,),
        in_specs=[
            pl.BlockSpec((NC, BR, HID), lambda i: (0, i, 0)),
            pl.BlockSpec((BR, HID), lambda i: (i, 0)),
            pl.BlockSpec((BR, 1), lambda i: (i, 0)),
            pl.BlockSpec((1, HID), lambda i: (0, 0)),
            pl.BlockSpec((BR, 1), lambda i: (i, 0)),
            pl.BlockSpec((HID, OUT), lambda i: (0, 0)),
            pl.BlockSpec((1, OUT), lambda i: (0, 0)),
        ],
        out_specs=pl.BlockSpec((G, OUT), lambda i: (0, 0)),
        out_shape=jax.ShapeDtypeStruct((G, OUT), jnp.float32),
        scratch_shapes=[
            pltpu.VMEM((G, HID), jnp.float32),
            pltpu.VMEM((G, 1), jnp.float32),
        ],
    )(aggp, ht2, dinv, b2, batch2d, Wh, bh)


def kernel(x, edge_index, batch, W1, b1, W2, b2, Wh, bh):
    src = edge_index[0]
    dst = edge_index[1]
    zeros1 = jnp.zeros((N,), jnp.float32)
    zrows = jnp.zeros((RPT, HID), jnp.float32)
    b1r = b1.reshape(1, HID)
    b2r = b2.reshape(1, HID)
    bhr = bh.reshape(1, OUT)
    batch2d = batch.reshape(N, 1)

    src3 = src.reshape(NW, NBLK, CPB, K)
    dst3 = dst.reshape(NW, NBLK, CPB, K)

    hists = _deg_kernel(dst, zeros1)
    ht1, dinv = _tc_mm1(hists.T, x, W1)
    agg1 = _agg_kernel(ht1, src3, dst3, zrows)
    ht2 = _tc_mid(agg1, ht1, dinv, b1r, W2)
    agg2 = _agg_kernel(ht2, src3, dst3, zrows)
    return _tc_pool(agg2, ht2, dinv, b2r, batch2d, Wh, bhr)
